# Initial kernel scaffold; baseline (speedup 1.0000x reference)
import functools

import jax
import jax.numpy as jnp
from jax import lax
from jax.experimental import pallas as pl
from jax.experimental.pallas import tpu as pltpu
from jax.experimental.pallas import tpu_sc as plsc

N = 10000
D = 128
NC = 2
NS = 16
NW = NC * NS
E2 = 640000
EPW = E2 // NW
C = 80
NCH = EPW // C
ROWS_PT = 640


def _sc_spmm_body(x_hbm, alpha_hbm, src_hbm, dst_hbm, val_hbm, out_hbm,
                  src_v, dst_v, val_v, alpha_v, rows0, rows1, scale_v, sem,
                  z_sh):
    cid = lax.axis_index("c")
    sid = lax.axis_index("s")
    wid = sid * NC + cid

    pltpu.sync_copy(src_hbm.at[wid], src_v)
    pltpu.sync_copy(dst_hbm.at[wid], dst_v)
    pltpu.sync_copy(val_hbm.at[wid], val_v)
    pltpu.sync_copy(alpha_hbm, alpha_v)

    zero = jnp.zeros((16,), jnp.float32)

    def _zrow(e, carry):
        for v in range(D // 16):
            rows0[e, pl.ds(v * 16, 16)] = zero
        return carry

    lax.fori_loop(0, C, _zrow, 0)
    start = sid * ROWS_PT
    for c in range(ROWS_PT // C):
        r0 = start + c * C

        @pl.when(r0 + C <= N)
        def _():
            pltpu.sync_copy(rows0, z_sh.at[pl.ds(r0, C)])

    plsc.subcore_barrier()

    w_lp = (wid < NS).astype(jnp.float32)

    bufs = (rows0, rows1)
    pltpu.make_async_copy(x_hbm.at[src_v.at[0]], rows0, sem).start()

    def _chunk(j, buf, nbuf):
        pltpu.make_async_copy(x_hbm.at[src_v.at[j]], buf, sem).wait()

        @pl.when(j + 1 < NCH)
        def _():
            pltpu.make_async_copy(x_hbm.at[src_v.at[j + 1]], nbuf, sem).start()

        for g in range(C // 16):
            dstv = dst_v[j, pl.ds(g * 16, 16)]
            av = plsc.load_gather(alpha_v, [dstv])
            vv = val_v[j, pl.ds(g * 16, 16)]
            w = w_lp * av + (1.0 - w_lp) * (1.0 - av)
            scale_v[pl.ds(g * 16, 16)] = vv * w

        def _erow(e, carry):
            s = scale_v[e]
            for v in range(D // 16):
                sl = pl.ds(v * 16, 16)
                buf[e, sl] = buf[e, sl] * s
            return carry

        lax.fori_loop(0, C, _erow, 0)

        pltpu.sync_copy(buf, z_sh.at[dst_v.at[j]], add=True)

    def _outer(jj, carry):
        for b2 in range(2):
            _chunk(jj * 2 + b2, bufs[b2], bufs[1 - b2])
        return carry

    lax.fori_loop(0, NCH // 2, _outer, 0)

    plsc.subcore_barrier()

    for c in range(ROWS_PT // C):
        r0 = start + c * C

        @pl.when(r0 + C <= N)
        def _():
            pltpu.sync_copy(z_sh.at[pl.ds(r0, C)], out_hbm.at[cid, pl.ds(r0, C)])


_sc_spmm = functools.partial(
    pl.kernel,
    out_type=jax.ShapeDtypeStruct((NC, N, D), jnp.float32),
    mesh=plsc.VectorSubcoreMesh(core_axis_name="c", subcore_axis_name="s",
                                num_cores=NC, num_subcores=NS),
    scratch_types=[
        pltpu.VMEM((NCH, C), jnp.int32),
        pltpu.VMEM((NCH, C), jnp.int32),
        pltpu.VMEM((NCH, C), jnp.float32),
        pltpu.VMEM((N,), jnp.float32),
        pltpu.VMEM((C, D), jnp.float32),
        pltpu.VMEM((C, D), jnp.float32),
        pltpu.VMEM((C,), jnp.float32),
        pltpu.SemaphoreType.DMA,
        pltpu.VMEM_SHARED((N, D), jnp.float32),
    ],
)(_sc_spmm_body)


def _alpha_body(x_ref, tw_ref, tb_ref, o_ref):
    t = jnp.sum(x_ref[...] * tw_ref[...], axis=1, keepdims=True) + tb_ref[0, 0]
    o_ref[...] = 1.0 / (1.0 + jnp.exp(-t))


def _alpha_tc(x, theta_w, theta_b):
    blk = 400
    grid = N // blk
    return pl.pallas_call(
        _alpha_body,
        grid=(grid,),
        in_specs=[
            pl.BlockSpec((blk, D), lambda i: (i, 0)),
            pl.BlockSpec((1, D), lambda i: (0, 0)),
            pl.BlockSpec((1, 1), lambda i: (0, 0)),
        ],
        out_specs=pl.BlockSpec((blk, 1), lambda i: (i, 0)),
        out_shape=jax.ShapeDtypeStruct((N, 1), jnp.float32),
    )(x, theta_w.reshape(1, D), theta_b.reshape(1, 1))


def _out_body(p_ref, w_ref, b_ref, o_ref):
    z = p_ref[0] + p_ref[1]
    o_ref[...] = jnp.maximum(
        jnp.dot(z, w_ref[...], preferred_element_type=jnp.float32) + b_ref[...],
        0.0)


def _out_tc(parts, W, b):
    blk = 2000
    grid = N // blk
    return pl.pallas_call(
        _out_body,
        grid=(grid,),
        in_specs=[
            pl.BlockSpec((NC, blk, D), lambda i: (0, i, 0)),
            pl.BlockSpec((D, D), lambda i: (0, 0)),
            pl.BlockSpec((1, D), lambda i: (0, 0)),
        ],
        out_specs=pl.BlockSpec((blk, D), lambda i: (i, 0)),
        out_shape=jax.ShapeDtypeStruct((N, D), jnp.float32),
    )(parts, W, b.reshape(1, D))


def kernel(x, theta_w, theta_b, W, b, vals_lp, src_lp, dst_lp,
           vals_hp, src_hp, dst_hp):
    alpha = _alpha_tc(x, theta_w, theta_b)
    src = jnp.concatenate([src_lp, src_hp]).reshape(NW, NCH, C)
    dst = jnp.concatenate([dst_lp, dst_hp]).reshape(NW, NCH, C)
    val = jnp.concatenate([vals_lp, vals_hp]).reshape(NW, NCH, C)
    parts = _sc_spmm(x, alpha.reshape(N), src, dst, val)
    out = _out_tc(parts, W, b)
    return out, alpha


# speedup vs baseline: 3.2166x; 3.2166x over previous
import functools

import jax
import jax.numpy as jnp
from jax import lax
from jax.experimental import pallas as pl
from jax.experimental.pallas import tpu as pltpu
from jax.experimental.pallas import tpu_sc as plsc

N = 10000
NP = 10240
D = 128
NC = 2
NS = 16
NW = NC * NS
E = 320000
C = 128
EPW = 20480
NCH = EPW // C
ROWS_PT = NP // NS


def _sc_spmm_body(x_hbm, alpha_hbm, edges_hbm, out_hbm,
                  alpha_v, rows0, rows1, ebuf, scale_v,
                  sem_g, sem_e, sem_s, z_sh):
    cid = lax.axis_index("c")
    sid = lax.axis_index("s")
    wid = sid * NC + cid

    pltpu.sync_copy(alpha_hbm, alpha_v)

    zero = jnp.zeros((16,), jnp.float32)

    def _zrow(e, carry):
        for v in range(D // 16):
            rows0[e, pl.ds(v * 16, 16)] = zero
        return carry

    lax.fori_loop(0, C, _zrow, 0)
    start = sid * ROWS_PT
    for c in range(ROWS_PT // C):
        pltpu.sync_copy(rows0, z_sh.at[pl.ds(start + c * C, C)])

    plsc.subcore_barrier()

    w_lp = jnp.full((16,), (wid < NS).astype(jnp.float32))
    w_hp = 1.0 - w_lp
    bufs = (rows0, rows1)

    pltpu.async_copy(edges_hbm.at[wid, 0], ebuf.at[pl.ds(0, 3)], sem_e)
    pltpu.make_async_copy(edges_hbm.at[wid, 0], ebuf.at[pl.ds(0, 3)], sem_e).wait()
    pltpu.async_copy(x_hbm.at[ebuf.at[0]], rows0, sem_g)
    pltpu.async_copy(edges_hbm.at[wid, 1], ebuf.at[pl.ds(3, 3)], sem_e)

    def _step(j, u):
        rows_b = bufs[u % 2]
        rows_nb = bufs[1 - u % 2]
        slot, nslot, nnslot, pslot = u, (u + 1) % 4, (u + 2) % 4, (u - 1) % 4

        pltpu.make_async_copy(x_hbm.at[ebuf.at[3 * slot]], rows_b, sem_g).wait()

        @pl.when(j > 0)
        def _():
            pltpu.make_async_copy(
                rows_nb, z_sh.at[ebuf.at[3 * pslot + 1]], sem_s).wait()

        @pl.when(j + 1 < NCH)
        def _():
            pltpu.make_async_copy(
                edges_hbm.at[wid, j + 1], ebuf.at[pl.ds(3 * nslot, 3)],
                sem_e).wait()
            pltpu.async_copy(x_hbm.at[ebuf.at[3 * nslot]], rows_nb, sem_g)

        @pl.when(j + 2 < NCH)
        def _():
            pltpu.async_copy(
                edges_hbm.at[wid, j + 2], ebuf.at[pl.ds(3 * nnslot, 3)], sem_e)

        for g in range(C // 16):
            sl = pl.ds(g * 16, 16)
            dstv = ebuf[3 * slot + 1, sl]
            av = plsc.load_gather(alpha_v, [dstv])
            vv = plsc.bitcast(ebuf[3 * slot + 2, sl], jnp.float32)
            scale_v[sl] = vv * (w_lp * av + w_hp * (1.0 - av))

        def _erow(e, carry):
            s16 = plsc.load_gather(scale_v, [jnp.full((16,), e, jnp.int32)])
            for v in range(D // 16):
                sl = pl.ds(v * 16, 16)
                rows_b[e, sl] = rows_b[e, sl] * s16
            return carry

        lax.fori_loop(0, C, _erow, 0)

        pltpu.async_copy(rows_b, z_sh.at[ebuf.at[3 * slot + 1]], sem_s, add=True)

    def _outer(jj, carry):
        for u in range(4):
            _step(jj * 4 + u, u)
        return carry

    lax.fori_loop(0, NCH // 4, _outer, 0)
    pltpu.make_async_copy(
        bufs[(NCH - 1) % 2], z_sh.at[ebuf.at[3 * 3 + 1]], sem_s).wait()

    plsc.subcore_barrier()

    for c in range(ROWS_PT // C):
        r0 = start + c * C
        pltpu.sync_copy(z_sh.at[pl.ds(r0, C)], out_hbm.at[cid, pl.ds(r0, C)])


_sc_spmm = functools.partial(
    pl.kernel,
    out_type=jax.ShapeDtypeStruct((NC, NP, D), jnp.float32),
    mesh=plsc.VectorSubcoreMesh(core_axis_name="c", subcore_axis_name="s",
                                num_cores=NC, num_subcores=NS),
    compiler_params=pltpu.CompilerParams(needs_layout_passes=False),
    scratch_types=[
        pltpu.VMEM((N,), jnp.float32),
        pltpu.VMEM((C, D), jnp.float32),
        pltpu.VMEM((C, D), jnp.float32),
        pltpu.VMEM((12, C), jnp.int32),
        pltpu.VMEM((C,), jnp.float32),
        pltpu.SemaphoreType.DMA,
        pltpu.SemaphoreType.DMA,
        pltpu.SemaphoreType.DMA,
        pltpu.VMEM_SHARED((NP, D), jnp.float32),
    ],
)(_sc_spmm_body)


def _alpha_body(x_ref, tw_ref, tb_ref, o_ref):
    t = jnp.sum(x_ref[...] * tw_ref[...], axis=1, keepdims=True) + tb_ref[0, 0]
    o_ref[...] = 1.0 / (1.0 + jnp.exp(-t))


def _alpha_tc(x, theta_w, theta_b):
    blk = 400
    return pl.pallas_call(
        _alpha_body,
        grid=(N // blk,),
        in_specs=[
            pl.BlockSpec((blk, D), lambda i: (i, 0)),
            pl.BlockSpec((1, D), lambda i: (0, 0)),
            pl.BlockSpec((1, 1), lambda i: (0, 0)),
        ],
        out_specs=pl.BlockSpec((blk, 1), lambda i: (i, 0)),
        out_shape=jax.ShapeDtypeStruct((N, 1), jnp.float32),
    )(x, theta_w.reshape(1, D), theta_b.reshape(1, 1))


def _out_body(p_ref, w_ref, b_ref, o_ref):
    z = p_ref[0] + p_ref[1]
    o_ref[...] = jnp.maximum(
        jnp.dot(z, w_ref[...], preferred_element_type=jnp.float32) + b_ref[...],
        0.0)


def _out_tc(parts, W, b):
    blk = 2000
    return pl.pallas_call(
        _out_body,
        grid=(N // blk,),
        in_specs=[
            pl.BlockSpec((NC, blk, D), lambda i: (0, i, 0)),
            pl.BlockSpec((D, D), lambda i: (0, 0)),
            pl.BlockSpec((1, D), lambda i: (0, 0)),
        ],
        out_specs=pl.BlockSpec((blk, D), lambda i: (i, 0)),
        out_shape=jax.ShapeDtypeStruct((N, D), jnp.float32),
    )(parts, W, b.reshape(1, D))


def _pack_edges(vals_lp, src_lp, dst_lp, vals_hp, src_hp, dst_hp):
    pad = NS * EPW - E

    def _one(src, dst, vals):
        src = jnp.concatenate([src, jnp.zeros((pad,), jnp.int32)])
        dst = jnp.concatenate([dst, jnp.zeros((pad,), jnp.int32)])
        vals = jnp.concatenate([vals, jnp.zeros((pad,), jnp.float32)])
        vbits = jax.lax.bitcast_convert_type(vals, jnp.int32)
        return jnp.stack(
            [src.reshape(NS, NCH, C), dst.reshape(NS, NCH, C),
             vbits.reshape(NS, NCH, C)], axis=2)

    return jnp.concatenate(
        [_one(src_lp, dst_lp, vals_lp), _one(src_hp, dst_hp, vals_hp)], axis=0)


def kernel(x, theta_w, theta_b, W, b, vals_lp, src_lp, dst_lp,
           vals_hp, src_hp, dst_hp):
    alpha = _alpha_tc(x, theta_w, theta_b)
    edges = _pack_edges(vals_lp, src_lp, dst_lp, vals_hp, src_hp, dst_hp)
    parts = _sc_spmm(x, alpha.reshape(N), edges)
    out = _out_tc(parts, W, b)
    return out, alpha
